# baseline (speedup 1.0000x reference)
import functools

import jax
import jax.numpy as jnp
from jax.experimental import pallas as pl
from jax.experimental.pallas import tpu as pltpu

N_U = 50000
N_I = 50000
D = 128
NNZ_POS = 600000
NNZ_NEG = 300000
NNZ_II = 600000
B = 16384
ITEM_ITEM_WEIGHT = 1.0
LAMBDA_REG = 1e-06

_ROW_BLK = 2000
_N_PAD = 50000


def _dense_update_body(msg_ref, e_ref, w_ref, b_ref, out_ref):
    x = msg_ref[...] + e_ref[...]
    y = jnp.dot(x, w_ref[...].T, preferred_element_type=jnp.float32) + b_ref[...]
    out_ref[...] = jnp.where(y >= 0, y, 0.2 * y)


def _dense_update(msg, e, w, b):
    n = msg.shape[0]
    grid = n // _ROW_BLK
    return pl.pallas_call(
        _dense_update_body,
        grid=(grid,),
        in_specs=[
            pl.BlockSpec((_ROW_BLK, D), lambda i: (i, 0)),
            pl.BlockSpec((_ROW_BLK, D), lambda i: (i, 0)),
            pl.BlockSpec((D, D), lambda i: (0, 0)),
            pl.BlockSpec((D,), lambda i: (0,)),
        ],
        out_specs=pl.BlockSpec((_ROW_BLK, D), lambda i: (i, 0)),
        out_shape=jax.ShapeDtypeStruct((n, D), jnp.float32),
    )(msg, e, w, b)


def _loss_body(u_ref, i_ref, lab_ref, logits_ref, loss_ref):
    step = pl.program_id(0)
    u = u_ref[...]
    v = i_ref[...]
    logits = jnp.sum(u * v, axis=-1)
    logits_ref[...] = logits[None, :]
    lab = lab_ref[...][0]
    bce = jnp.maximum(logits, 0.0) - logits * lab + jnp.log1p(jnp.exp(-jnp.abs(logits)))
    reg = jnp.sum(u * u) + jnp.sum(v * v)
    part = jnp.sum(bce) / B + LAMBDA_REG * reg

    @pl.when(step == 0)
    def _init():
        loss_ref[0, 0] = 0.0

    loss_ref[0, 0] += part


def _loss(u_emb, i_emb, labels):
    blk = 2048
    grid = B // blk
    logits, loss = pl.pallas_call(
        _loss_body,
        grid=(grid,),
        in_specs=[
            pl.BlockSpec((blk, D), lambda i: (i, 0)),
            pl.BlockSpec((blk, D), lambda i: (i, 0)),
            pl.BlockSpec((1, blk), lambda i: (0, i)),
        ],
        out_specs=[
            pl.BlockSpec((1, blk), lambda i: (0, i)),
            pl.BlockSpec(memory_space=pltpu.SMEM),
        ],
        out_shape=[
            jax.ShapeDtypeStruct((1, B), jnp.float32),
            jax.ShapeDtypeStruct((1, 1), jnp.float32),
        ],
    )(u_emb, i_emb, labels.reshape(1, B))
    return logits.reshape(B), loss[0, 0]


def _spmm(idx, val, x, n_rows):
    return jax.ops.segment_sum(val[:, None] * x[idx[1]], idx[0], num_segments=n_rows)


def _spmm_t(idx, val, x, n_cols):
    return jax.ops.segment_sum(val[:, None] * x[idx[0]], idx[1], num_segments=n_cols)


def kernel(uids, iids, labels, E_u_0, E_i_0, W0, b0, W1, b1, W2, b2,
           pos_idx, pos_val, neg_idx, neg_val, ii_idx, ii_val):
    E_u = E_u_0
    E_i = E_i_0
    for W, b in ((W0, b0), (W1, b1), (W2, b2)):
        msg_u = _spmm(pos_idx, pos_val, E_i, N_U) - _spmm(neg_idx, neg_val, E_i, N_U)
        msg_i = _spmm_t(pos_idx, pos_val, E_u, N_I) - _spmm_t(neg_idx, neg_val, E_u, N_I)
        msg_i = msg_i + ITEM_ITEM_WEIGHT * _spmm(ii_idx, ii_val, E_i, N_I)
        E_u = _dense_update(msg_u, E_u, W, b)
        E_i = _dense_update(msg_i, E_i, W, b)
    E_u_n = E_u / jnp.maximum(jnp.linalg.norm(E_u, axis=-1, keepdims=True), 1e-12)
    u_emb = E_u_n[uids]
    i_emb = E_i[iids]
    logits, loss = _loss(u_emb, i_emb, labels)
    return (loss, logits)


# speedup vs baseline: 2.0184x; 2.0184x over previous
import functools

import jax
import jax.numpy as jnp
from jax import lax
from jax.experimental import pallas as pl
from jax.experimental.pallas import tpu as pltpu
from jax.experimental.pallas import tpu_sc as plsc

N_U = 50000
N_I = 50000
D = 128
NNZ_POS = 600000
NNZ_NEG = 300000
NNZ_II = 600000
B = 16384
ITEM_ITEM_WEIGHT = 1.0
LAMBDA_REG = 1e-06

NC = 2
NS = 16
NW = NC * NS

CHUNK = 11520
NCH = 5
NPAD = CHUNK * NCH
CAP = 28416
G = 128
STRIPE = CHUNK // NS
ZROWS = 80

_mesh = plsc.VectorSubcoreMesh(
    core_axis_name="c", subcore_axis_name="s", num_cores=NC, num_subcores=NS)


def _iota16():
    return lax.iota(jnp.int32, 16)


def _nb(e):
    return (e + FB - 1) // FB


def _skip(e):
    return (_nb(e) - 1) * FB - (e - FB)


def _prefix_packed(x, iota):
    dnums = lax.GatherDimensionNumbers(offset_dims=(),
                                       collapsed_slice_dims=(0,),
                                       start_index_map=(0,))
    for t in (1, 2, 4, 8):
        idx = jnp.maximum(iota - t, 0)
        g = lax.gather(x, idx[:, None], dnums, (1,),
                       mode=lax.GatherScatterMode.PROMISE_IN_BOUNDS)
        x = x + jnp.where(iota >= t, g, 0)
    return x


GB = 128
TOT = 3 * NCH * NW * CAP
TRASH = TOT


def _bucket_body(pos_r, pos_c, pos_val, neg_r, neg_c, neg_val,
                 ii_r, ii_c, ii_val,
                 keys, vals, counts,
                 dbuf, sbuf, vbuf, kstage, vstage, didx, cstage, sem):
    cid = lax.axis_index("c")
    sid = lax.axis_index("s")
    w = sid * NC + cid
    iota = _iota16()
    ones = jnp.ones((16,), jnp.int32)

    def scan(dst_ref, src_ref, val_ref, sign, j, fills):
        e = dst_ref.shape[0]
        nb = (e + GB - 1) // GB
        skip_last = (nb - 1) * GB - (e - GB)
        t_max = (nb + NW - 1) // NW
        rbase0 = (j * NCH * NW + w) * CAP

        def outer(t, fills):
            f0, f1, f2, f3, f4 = fills
            i = w + NW * t
            valid = i < nb
            ic = jnp.minimum(i, nb - 1)
            is_last = ic == nb - 1
            off = pl.multiple_of(jnp.where(is_last, e - GB, ic * GB), 8)
            skip = jnp.where(is_last, skip_last, 0)
            skip = jnp.where(valid, skip, GB)
            c1 = pltpu.async_copy(dst_ref.at[pl.ds(off, GB)], dbuf, sem)
            c2 = pltpu.async_copy(src_ref.at[pl.ds(off, GB)], sbuf, sem)
            c3 = pltpu.async_copy(val_ref.at[pl.ds(off, GB)], vbuf, sem)
            c1.wait(); c2.wait(); c3.wait()

            def ivec(v, fcarry):
                f0, f1, f2, f3, f4 = fcarry
                d = dbuf[pl.ds(v * 16, 16)]
                s = sbuf[pl.ds(v * 16, 16)]
                x = vbuf[pl.ds(v * 16, 16)]
                lanes = iota + v * 16
                mval = lanes >= skip
                bvec = (jnp.where(d >= CHUNK, 1, 0)
                        + jnp.where(d >= 2 * CHUNK, 1, 0)
                        + jnp.where(d >= 3 * CHUNK, 1, 0)
                        + jnp.where(d >= 4 * CHUNK, 1, 0))
                drel = d - bvec * CHUNK
                key = lax.bitwise_or(lax.shift_left(drel, 16), s)
                sval = x * sign
                sh = bvec * 6
                oneh = jnp.where(mval, lax.shift_left(ones, sh), 0)
                pref = _prefix_packed(oneh, iota)
                rank = lax.bitwise_and(lax.shift_right_logical(pref, sh), 63)
                fv = jnp.where(bvec == 0, f0,
                               jnp.where(bvec == 1, f1,
                                         jnp.where(bvec == 2, f2,
                                                   jnp.where(bvec == 3, f3,
                                                             f4))))
                di = rbase0 + bvec * (NW * CAP) + fv + rank - 1
                didx[pl.ds(v * 16, 16)] = jnp.where(mval, di, TRASH)
                kstage[pl.ds(v * 16, 16)] = key
                vstage[pl.ds(v * 16, 16)] = sval
                p15 = pref[15]
                f0 = f0 + lax.bitwise_and(p15, 63)
                f1 = f1 + lax.bitwise_and(lax.shift_right_logical(p15, 6), 63)
                f2 = f2 + lax.bitwise_and(lax.shift_right_logical(p15, 12), 63)
                f3 = f3 + lax.bitwise_and(lax.shift_right_logical(p15, 18), 63)
                f4 = f4 + lax.bitwise_and(lax.shift_right_logical(p15, 24), 63)
                return (f0, f1, f2, f3, f4)

            f0, f1, f2, f3, f4 = lax.fori_loop(0, GB // 16, ivec,
                                               (f0, f1, f2, f3, f4))
            pltpu.sync_copy(kstage, keys.at[didx])
            pltpu.sync_copy(vstage, vals.at[didx])
            return (f0, f1, f2, f3, f4)

        return lax.fori_loop(0, t_max, outer, fills)

    cnt = jnp.zeros((16,), jnp.int32)
    jobs = (
        (0, ((pos_r, pos_c, pos_val, 1.0), (neg_r, neg_c, neg_val, -1.0))),
        (1, ((pos_c, pos_r, pos_val, 1.0), (neg_c, neg_r, neg_val, -1.0))),
        (2, ((ii_r, ii_c, ii_val, 1.0),)),
    )
    for j, subscans in jobs:
        fills = (jnp.int32(0),) * 5
        for dst_ref, src_ref, val_ref, sign in subscans:
            fills = scan(dst_ref, src_ref, val_ref, sign, j, fills)
        for b in range(NCH):
            cnt = jnp.where(iota == j * NCH + b, fills[b], cnt)
    cstage[...] = cnt
    pltpu.sync_copy(cstage, counts.at[pl.ds(pl.multiple_of(w * 16, 16), 16)])


_bucket_kernel = functools.partial(
    pl.kernel,
    out_type=(
        jax.ShapeDtypeStruct((TOT + 8,), jnp.int32),
        jax.ShapeDtypeStruct((TOT + 8,), jnp.float32),
        jax.ShapeDtypeStruct((NW * 16,), jnp.int32),
    ),
    mesh=_mesh,
    scratch_types=[
        pltpu.VMEM((GB,), jnp.int32),
        pltpu.VMEM((GB,), jnp.int32),
        pltpu.VMEM((GB,), jnp.float32),
        pltpu.VMEM((GB,), jnp.int32),
        pltpu.VMEM((GB,), jnp.float32),
        pltpu.VMEM((GB,), jnp.int32),
        pltpu.VMEM((16,), jnp.int32),
        pltpu.SemaphoreType.DMA,
    ],
)(_bucket_body)


def _spmm_body(keys, vals, counts, eu, ei, msgu, msgi,
               kbuf, vbuf, sidx, didx, rows, zbuf, cntv, accum, sem):
    cid = lax.axis_index("c")
    sid = lax.axis_index("s")
    iota = _iota16()

    def zinit(r, _):
        for c in range(D // 16):
            zbuf[r, pl.ds(c * 16, 16)] = jnp.zeros((16,), jnp.float32)
        return 0
    lax.fori_loop(0, ZROWS, zinit, 0)

    def zero_accum():
        for k in range(STRIPE // ZROWS):
            pltpu.sync_copy(
                zbuf,
                accum.at[pl.ds(pl.multiple_of(sid * STRIPE + k * ZROWS, 16),
                               ZROWS)])

    def process_job(j, table, b):
        def per_w(wi, _):
            w = sid * NC + wi
            pltpu.sync_copy(
                counts.at[pl.ds(pl.multiple_of(w * 16, 16), 16)],
                cntv.at[pl.ds(0, 16)])
            slot = j * NCH + b
            n = cntv[pl.ds(slot, 16)][0]
            nbatch = (n + G - 1) // G
            rbase = ((j * NCH + b) * NW + w) * CAP

            def batch(g, _):
                base = pl.multiple_of(rbase + g * G, G)
                c1 = pltpu.async_copy(keys.at[pl.ds(base, G)], kbuf, sem)
                c2 = pltpu.async_copy(vals.at[pl.ds(base, G)], vbuf, sem)
                c1.wait(); c2.wait()
                gbase = g * G
                for u in range(G // 16):
                    ok = (gbase + u * 16 + iota) < n
                    k = kbuf[pl.ds(u * 16, 16)]
                    sidx[pl.ds(u * 16, 16)] = jnp.where(
                        ok, lax.bitwise_and(k, 0xFFFF), 0)
                    didx[pl.ds(u * 16, 16)] = jnp.where(
                        ok, lax.shift_right_logical(k, 16), CHUNK)
                    vv = vbuf[pl.ds(u * 16, 16)]
                    vbuf[pl.ds(u * 16, 16)] = jnp.where(ok, vv, 0.0)
                pltpu.async_copy(table.at[sidx], rows, sem).wait()
                for q in range(G // 16):
                    vv = vbuf[pl.ds(q * 16, 16)]
                    for e2 in range(16):
                        e = q * 16 + e2
                        s = vv[e2]
                        for c in range(D // 16):
                            rows[e, pl.ds(c * 16, 16)] = (
                                rows[e, pl.ds(c * 16, 16)] * s)
                pltpu.sync_copy(rows, accum.at[didx], add=True)
                return 0

            lax.fori_loop(0, nbatch, batch, 0)
            return 0

        lax.fori_loop(0, NC, per_w, 0)

    def chunk_iter(t, _):
        b = cid + NC * t

        @pl.when(b < NCH)
        def _do():
            do_chunk(b)

        return 0

    def do_chunk(b):
        lo = pl.multiple_of(b * CHUNK, 16)
        zero_accum()
        plsc.subcore_barrier()
        process_job(jnp.int32(0), ei, b)
        plsc.subcore_barrier()
        pltpu.sync_copy(
            accum.at[pl.ds(pl.multiple_of(sid * STRIPE, 16), STRIPE)],
            msgu.at[pl.ds(pl.multiple_of(lo + sid * STRIPE, 16), STRIPE)])
        zero_accum()
        plsc.subcore_barrier()
        process_job(jnp.int32(1), eu, b)
        process_job(jnp.int32(2), ei, b)
        plsc.subcore_barrier()
        pltpu.sync_copy(
            accum.at[pl.ds(pl.multiple_of(sid * STRIPE, 16), STRIPE)],
            msgi.at[pl.ds(pl.multiple_of(lo + sid * STRIPE, 16), STRIPE)])
        plsc.subcore_barrier()

    lax.fori_loop(0, (NCH + NC - 1) // NC, chunk_iter, 0)


_spmm_kernel = functools.partial(
    pl.kernel,
    out_type=(
        jax.ShapeDtypeStruct((NPAD, D), jnp.float32),
        jax.ShapeDtypeStruct((NPAD, D), jnp.float32),
    ),
    mesh=_mesh,
    scratch_types=[
        pltpu.VMEM((G,), jnp.int32),
        pltpu.VMEM((G,), jnp.float32),
        pltpu.VMEM((G,), jnp.int32),
        pltpu.VMEM((G,), jnp.int32),
        pltpu.VMEM((G, D), jnp.float32),
        pltpu.VMEM((ZROWS, D), jnp.float32),
        pltpu.VMEM((32,), jnp.int32),
        pltpu.VMEM_SHARED((CHUNK + 8, D), jnp.float32),
        pltpu.SemaphoreType.DMA,
    ],
)(_spmm_body)


def _gather_body(eu, ei, uids, iids, u_out, i_out, idxb, rows, sem):
    cid = lax.axis_index("c")
    sid = lax.axis_index("s")
    wid = sid * NC + cid
    per = B // NW
    base = wid * per
    for table, ids, out in ((eu, uids, u_out), (ei, iids, i_out)):
        for k in range(per // G):
            o = pl.multiple_of(base + k * G, G)
            pltpu.sync_copy(ids.at[pl.ds(o, G)], idxb)
            pltpu.async_copy(table.at[idxb], rows, sem).wait()
            pltpu.sync_copy(rows, out.at[pl.ds(o, G)])


_gather_kernel = functools.partial(
    pl.kernel,
    out_type=(
        jax.ShapeDtypeStruct((B, D), jnp.float32),
        jax.ShapeDtypeStruct((B, D), jnp.float32),
    ),
    mesh=_mesh,
    scratch_types=[
        pltpu.VMEM((G,), jnp.int32),
        pltpu.VMEM((G, D), jnp.float32),
        pltpu.SemaphoreType.DMA,
    ],
)(_gather_body)


_ROW_BLK = 1920


def _dense_update_body(msg_ref, e_ref, w_ref, b_ref, out_ref):
    x = msg_ref[...] + e_ref[...]
    y = jnp.dot(x, w_ref[...].T, preferred_element_type=jnp.float32) + b_ref[...]
    out_ref[...] = jnp.where(y >= 0, y, 0.2 * y)


def _dense_update(msg, e, w, b):
    n = msg.shape[0]
    return pl.pallas_call(
        _dense_update_body,
        grid=(n // _ROW_BLK,),
        in_specs=[
            pl.BlockSpec((_ROW_BLK, D), lambda i: (i, 0)),
            pl.BlockSpec((_ROW_BLK, D), lambda i: (i, 0)),
            pl.BlockSpec((D, D), lambda i: (0, 0)),
            pl.BlockSpec((D,), lambda i: (0,)),
        ],
        out_specs=pl.BlockSpec((_ROW_BLK, D), lambda i: (i, 0)),
        out_shape=jax.ShapeDtypeStruct((n, D), jnp.float32),
    )(msg, e, w, b)


def _loss_body(u_ref, i_ref, lab_ref, logits_ref, loss_ref):
    step = pl.program_id(0)
    u = u_ref[...]
    nrm = jnp.sqrt(jnp.sum(u * u, axis=-1, keepdims=True))
    u = u / jnp.maximum(nrm, 1e-12)
    v = i_ref[...]
    logits = jnp.sum(u * v, axis=-1)
    logits_ref[...] = logits[None, :]
    lab = lab_ref[...][0]
    bce = jnp.maximum(logits, 0.0) - logits * lab + jnp.log1p(jnp.exp(-jnp.abs(logits)))
    reg = jnp.sum(u * u) + jnp.sum(v * v)
    part = jnp.sum(bce) / B + LAMBDA_REG * reg

    @pl.when(step == 0)
    def _init():
        loss_ref[0, 0] = 0.0

    loss_ref[0, 0] += part


def _loss(u_emb, i_emb, labels):
    blk = 2048
    logits, loss = pl.pallas_call(
        _loss_body,
        grid=(B // blk,),
        in_specs=[
            pl.BlockSpec((blk, D), lambda i: (i, 0)),
            pl.BlockSpec((blk, D), lambda i: (i, 0)),
            pl.BlockSpec((1, blk), lambda i: (0, i)),
        ],
        out_specs=[
            pl.BlockSpec((1, blk), lambda i: (0, i)),
            pl.BlockSpec(memory_space=pltpu.SMEM),
        ],
        out_shape=[
            jax.ShapeDtypeStruct((1, B), jnp.float32),
            jax.ShapeDtypeStruct((1, 1), jnp.float32),
        ],
    )(u_emb, i_emb, labels.reshape(1, B))
    return logits.reshape(B), loss[0, 0]


def kernel(uids, iids, labels, E_u_0, E_i_0, W0, b0, W1, b1, W2, b2,
           pos_idx, pos_val, neg_idx, neg_val, ii_idx, ii_val):
    keys, vals, counts = _bucket_kernel(
        pos_idx[0], pos_idx[1], pos_val,
        neg_idx[0], neg_idx[1], neg_val,
        ii_idx[0], ii_idx[1], ii_val)
    E_u = jnp.pad(E_u_0, ((0, NPAD - N_U), (0, 0)))
    E_i = jnp.pad(E_i_0, ((0, NPAD - N_I), (0, 0)))
    for W, b in ((W0, b0), (W1, b1), (W2, b2)):
        msg_u, msg_i = _spmm_kernel(keys, vals, counts, E_u, E_i)
        E_u = _dense_update(msg_u, E_u, W, b)
        E_i = _dense_update(msg_i, E_i, W, b)
    u_emb, i_emb = _gather_kernel(E_u, E_i, uids, iids)
    logits, loss = _loss(u_emb, i_emb, labels)
    return (loss, logits)


# speedup vs baseline: 3.7294x; 1.8477x over previous
import functools

import jax
import jax.numpy as jnp
from jax import lax
from jax.experimental import pallas as pl
from jax.experimental.pallas import tpu as pltpu
from jax.experimental.pallas import tpu_sc as plsc

N_U = 50000
N_I = 50000
D = 128
NNZ_POS = 600000
NNZ_NEG = 300000
NNZ_II = 600000
B = 16384
ITEM_ITEM_WEIGHT = 1.0
LAMBDA_REG = 1e-06

NC = 2
NS = 16
NW = NC * NS

CHUNK = 11520
NCH = 5
NPAD = CHUNK * NCH
CAP = 28416
G = 128
STRIPE = CHUNK // NS
ZROWS = 80

_mesh = plsc.VectorSubcoreMesh(
    core_axis_name="c", subcore_axis_name="s", num_cores=NC, num_subcores=NS)


def _iota16():
    return lax.iota(jnp.int32, 16)


def _nb(e):
    return (e + FB - 1) // FB


def _skip(e):
    return (_nb(e) - 1) * FB - (e - FB)


_DNUMS = lax.GatherDimensionNumbers(offset_dims=(),
                                    collapsed_slice_dims=(0,),
                                    start_index_map=(0,))


def _dgather(x, idx):
    return lax.gather(x, idx[:, None], _DNUMS, (1,),
                      mode=lax.GatherScatterMode.PROMISE_IN_BOUNDS)


def _prefix_packed(x, iota):
    for t in (1, 2, 4, 8):
        g = _dgather(x, jnp.maximum(iota - t, 0))
        x = x + jnp.where(iota >= t, g, 0)
    return x


GB = 128
RING = 512
FB = 256
TOT = 3 * NCH * NW * CAP
TRASH = TOT


def _bucket_body(pos_r, pos_c, pos_val, neg_r, neg_c, neg_val,
                 ii_r, ii_c, ii_val,
                 keys, vals, counts,
                 dbuf, sbuf, vbuf, ring_k, ring_v, cstage, sem):
    cid = lax.axis_index("c")
    sid = lax.axis_index("s")
    w = sid * NC + cid
    iota = _iota16()
    ones = jnp.ones((16,), jnp.int32)

    def scan(dst_ref, src_ref, val_ref, sign, j, state):
        e = dst_ref.shape[0]
        nb = (e + GB - 1) // GB
        skip_last = (nb - 1) * GB - (e - GB)
        t_max = (nb + NW - 1) // NW
        rbase0 = (j * NCH * NW + w) * CAP

        def outer(t, state):
            fills, flushed = state
            i = w + NW * t
            valid = i < nb
            ic = jnp.minimum(i, nb - 1)
            is_last = ic == nb - 1
            off = pl.multiple_of(jnp.where(is_last, e - GB, ic * GB), 8)
            skip = jnp.where(is_last, skip_last, 0)
            skip = jnp.where(valid, skip, GB)
            c1 = pltpu.async_copy(dst_ref.at[pl.ds(off, GB)], dbuf, sem)
            c2 = pltpu.async_copy(src_ref.at[pl.ds(off, GB)], sbuf, sem)
            c3 = pltpu.async_copy(val_ref.at[pl.ds(off, GB)], vbuf, sem)
            c1.wait(); c2.wait(); c3.wait()

            def ivec(v, state):
                fills, flushed = state
                fills = list(fills)
                flushed = list(flushed)
                d = dbuf[pl.ds(v * 16, 16)]
                s = sbuf[pl.ds(v * 16, 16)]
                x = vbuf[pl.ds(v * 16, 16)]
                lanes = iota + v * 16
                mval = lanes >= skip
                bvec = (jnp.where(d >= CHUNK, 1, 0)
                        + jnp.where(d >= 2 * CHUNK, 1, 0)
                        + jnp.where(d >= 3 * CHUNK, 1, 0)
                        + jnp.where(d >= 4 * CHUNK, 1, 0))
                drel = d - bvec * CHUNK
                key = lax.bitwise_or(lax.shift_left(drel, 16), s)
                sval = x * sign
                sh = bvec * 6
                oneh = jnp.where(mval, lax.shift_left(ones, sh), 0)
                pref = _prefix_packed(oneh, iota)
                p15 = pref[15]
                cnt_b = [lax.bitwise_and(
                    lax.shift_right_logical(p15, 6 * b), 63)
                    for b in range(NCH)]
                start_b = [jnp.int32(0)]
                for b in range(1, NCH):
                    start_b.append(start_b[b - 1] + cnt_b[b - 1])
                n_valid = start_b[NCH - 1] + cnt_b[NCH - 1]
                rank_in = lax.bitwise_and(lax.shift_right_logical(pref, sh),
                                          63) - 1
                start_v = jnp.where(bvec == 0, start_b[0],
                           jnp.where(bvec == 1, start_b[1],
                            jnp.where(bvec == 2, start_b[2],
                             jnp.where(bvec == 3, start_b[3], start_b[4]))))
                tot_incl = jnp.zeros((16,), jnp.int32)
                for b in range(NCH):
                    tot_incl = tot_incl + lax.bitwise_and(
                        lax.shift_right_logical(pref, 6 * b), 63)
                rank_g = jnp.where(mval, start_v + rank_in,
                                   n_valid + iota - tot_incl)
                perm = jnp.zeros((16,), jnp.int32)
                for l in range(16):
                    rl = rank_g[l]
                    perm = jnp.where(iota == rl, l, perm)
                sk = _dgather(key, perm)
                sv = _dgather(sval, perm)
                for b in range(NCH):
                    seg_idx = jnp.minimum(iota + start_b[b], 15)
                    segk = _dgather(sk, seg_idx)
                    segv = _dgather(sv, seg_idx)
                    rb = b * (RING + 16)
                    p = lax.bitwise_and(fills[b], RING - 1)
                    ring_k[pl.ds(rb + p, 16)] = segk
                    ring_v[pl.ds(rb + p, 16)] = segv

                    @pl.when(p >= RING - 15)
                    def _mirror(rb=rb):
                        ring_k[pl.ds(rb, 16)] = ring_k[pl.ds(rb + RING, 16)]
                        ring_v[pl.ds(rb, 16)] = ring_v[pl.ds(rb + RING, 16)]

                    f2 = fills[b] + cnt_b[b]
                    do_flush = (f2 - flushed[b]) >= FB

                    @pl.when(do_flush)
                    def _flush(rb=rb, b=b, fl=flushed[b]):
                        par = pl.multiple_of(
                            rb + lax.bitwise_and(fl, RING - 1), 8)
                        dst = pl.multiple_of(
                            rbase0 + b * (NW * CAP) + fl, 8)
                        pltpu.sync_copy(ring_k.at[pl.ds(par, FB)],
                                        keys.at[pl.ds(dst, FB)])
                        pltpu.sync_copy(ring_v.at[pl.ds(par, FB)],
                                        vals.at[pl.ds(dst, FB)])

                    fills[b] = f2
                    flushed[b] = jnp.where(do_flush, flushed[b] + FB,
                                           flushed[b])
                return (tuple(fills), tuple(flushed))

            return lax.fori_loop(0, GB // 16, ivec, (fills, flushed))

        return lax.fori_loop(0, t_max, outer, state)

    def drain(j, state):
        fills, flushed = state
        for b in range(NCH):
            @pl.when(fills[b] > flushed[b])
            def _fl(b=b, fl=flushed[b]):
                rb = b * (RING + 16)
                par = pl.multiple_of(rb + lax.bitwise_and(fl, RING - 1), 8)
                dst = pl.multiple_of(
                    (j * NCH * NW + w) * CAP + b * (NW * CAP) + fl, 8)
                pltpu.sync_copy(ring_k.at[pl.ds(par, FB)],
                                keys.at[pl.ds(dst, FB)])
                pltpu.sync_copy(ring_v.at[pl.ds(par, FB)],
                                vals.at[pl.ds(dst, FB)])

    cnt = jnp.zeros((16,), jnp.int32)
    jobs = (
        (0, ((pos_r, pos_c, pos_val, 1.0), (neg_r, neg_c, neg_val, -1.0))),
        (1, ((pos_c, pos_r, pos_val, 1.0), (neg_c, neg_r, neg_val, -1.0))),
        (2, ((ii_r, ii_c, ii_val, 1.0),)),
    )
    for j, subscans in jobs:
        state = ((jnp.int32(0),) * NCH, (jnp.int32(0),) * NCH)
        for dst_ref, src_ref, val_ref, sign in subscans:
            state = scan(dst_ref, src_ref, val_ref, sign, j, state)
        drain(j, state)
        for b in range(NCH):
            cnt = jnp.where(iota == j * NCH + b, state[0][b], cnt)
    cstage[...] = cnt
    pltpu.sync_copy(cstage, counts.at[pl.ds(pl.multiple_of(w * 16, 16), 16)])


_bucket_kernel = functools.partial(
    pl.kernel,
    out_type=(
        jax.ShapeDtypeStruct((TOT + 8,), jnp.int32),
        jax.ShapeDtypeStruct((TOT + 8,), jnp.float32),
        jax.ShapeDtypeStruct((NW * 16,), jnp.int32),
    ),
    mesh=_mesh,
    scratch_types=[
        pltpu.VMEM((GB,), jnp.int32),
        pltpu.VMEM((GB,), jnp.int32),
        pltpu.VMEM((GB,), jnp.float32),
        pltpu.VMEM((NCH * (RING + 16),), jnp.int32),
        pltpu.VMEM((NCH * (RING + 16),), jnp.float32),
        pltpu.VMEM((16,), jnp.int32),
        pltpu.SemaphoreType.DMA,
    ],
)(_bucket_body)


def _spmm_body(keys, vals, counts, eu, ei, msgu, msgi,
               kbuf, vbuf, sidx, didx, rows, zbuf, cntv, accum, sem):
    cid = lax.axis_index("c")
    sid = lax.axis_index("s")
    iota = _iota16()

    def zinit(r, _):
        for c in range(D // 16):
            zbuf[r, pl.ds(c * 16, 16)] = jnp.zeros((16,), jnp.float32)
        return 0
    lax.fori_loop(0, ZROWS, zinit, 0)

    def zero_accum():
        for k in range(STRIPE // ZROWS):
            pltpu.sync_copy(
                zbuf,
                accum.at[pl.ds(pl.multiple_of(sid * STRIPE + k * ZROWS, 16),
                               ZROWS)])

    def process_job(j, table, b):
        def per_w(wi, _):
            w = sid * NC + wi
            pltpu.sync_copy(
                counts.at[pl.ds(pl.multiple_of(w * 16, 16), 16)],
                cntv.at[pl.ds(0, 16)])
            slot = j * NCH + b
            n = cntv[pl.ds(slot, 16)][0]
            nbatch = (n + G - 1) // G
            rbase = ((j * NCH + b) * NW + w) * CAP

            def batch(g, _):
                base = pl.multiple_of(rbase + g * G, G)
                c1 = pltpu.async_copy(keys.at[pl.ds(base, G)], kbuf, sem)
                c2 = pltpu.async_copy(vals.at[pl.ds(base, G)], vbuf, sem)
                c1.wait(); c2.wait()
                gbase = g * G
                for u in range(G // 16):
                    ok = (gbase + u * 16 + iota) < n
                    k = kbuf[pl.ds(u * 16, 16)]
                    sidx[pl.ds(u * 16, 16)] = jnp.where(
                        ok, lax.bitwise_and(k, 0xFFFF), 0)
                    didx[pl.ds(u * 16, 16)] = jnp.where(
                        ok, lax.shift_right_logical(k, 16), CHUNK)
                    vv = vbuf[pl.ds(u * 16, 16)]
                    vbuf[pl.ds(u * 16, 16)] = jnp.where(ok, vv, 0.0)
                pltpu.async_copy(table.at[sidx], rows, sem).wait()
                for q in range(G // 16):
                    vv = vbuf[pl.ds(q * 16, 16)]
                    for e2 in range(16):
                        e = q * 16 + e2
                        s = vv[e2]
                        for c in range(D // 16):
                            rows[e, pl.ds(c * 16, 16)] = (
                                rows[e, pl.ds(c * 16, 16)] * s)
                pltpu.sync_copy(rows, accum.at[didx], add=True)
                return 0

            lax.fori_loop(0, nbatch, batch, 0)
            return 0

        lax.fori_loop(0, NC, per_w, 0)

    def chunk_iter(t, _):
        b = cid + NC * t

        @pl.when(b < NCH)
        def _do():
            do_chunk(b)

        return 0

    def do_chunk(b):
        lo = pl.multiple_of(b * CHUNK, 16)
        zero_accum()
        plsc.subcore_barrier()
        process_job(jnp.int32(0), ei, b)
        plsc.subcore_barrier()
        pltpu.sync_copy(
            accum.at[pl.ds(pl.multiple_of(sid * STRIPE, 16), STRIPE)],
            msgu.at[pl.ds(pl.multiple_of(lo + sid * STRIPE, 16), STRIPE)])
        zero_accum()
        plsc.subcore_barrier()
        process_job(jnp.int32(1), eu, b)
        process_job(jnp.int32(2), ei, b)
        plsc.subcore_barrier()
        pltpu.sync_copy(
            accum.at[pl.ds(pl.multiple_of(sid * STRIPE, 16), STRIPE)],
            msgi.at[pl.ds(pl.multiple_of(lo + sid * STRIPE, 16), STRIPE)])
        plsc.subcore_barrier()

    lax.fori_loop(0, (NCH + NC - 1) // NC, chunk_iter, 0)


_spmm_kernel = functools.partial(
    pl.kernel,
    out_type=(
        jax.ShapeDtypeStruct((NPAD, D), jnp.float32),
        jax.ShapeDtypeStruct((NPAD, D), jnp.float32),
    ),
    mesh=_mesh,
    scratch_types=[
        pltpu.VMEM((G,), jnp.int32),
        pltpu.VMEM((G,), jnp.float32),
        pltpu.VMEM((G,), jnp.int32),
        pltpu.VMEM((G,), jnp.int32),
        pltpu.VMEM((G, D), jnp.float32),
        pltpu.VMEM((ZROWS, D), jnp.float32),
        pltpu.VMEM((32,), jnp.int32),
        pltpu.VMEM_SHARED((CHUNK + 8, D), jnp.float32),
        pltpu.SemaphoreType.DMA,
    ],
)(_spmm_body)


def _gather_body(eu, ei, uids, iids, u_out, i_out, idxb, rows, sem):
    cid = lax.axis_index("c")
    sid = lax.axis_index("s")
    wid = sid * NC + cid
    per = B // NW
    base = wid * per
    for table, ids, out in ((eu, uids, u_out), (ei, iids, i_out)):
        for k in range(per // G):
            o = pl.multiple_of(base + k * G, G)
            pltpu.sync_copy(ids.at[pl.ds(o, G)], idxb)
            pltpu.async_copy(table.at[idxb], rows, sem).wait()
            pltpu.sync_copy(rows, out.at[pl.ds(o, G)])


_gather_kernel = functools.partial(
    pl.kernel,
    out_type=(
        jax.ShapeDtypeStruct((B, D), jnp.float32),
        jax.ShapeDtypeStruct((B, D), jnp.float32),
    ),
    mesh=_mesh,
    scratch_types=[
        pltpu.VMEM((G,), jnp.int32),
        pltpu.VMEM((G, D), jnp.float32),
        pltpu.SemaphoreType.DMA,
    ],
)(_gather_body)


_ROW_BLK = 1920


def _dense_update_body(msg_ref, e_ref, w_ref, b_ref, out_ref):
    x = msg_ref[...] + e_ref[...]
    y = jnp.dot(x, w_ref[...].T, preferred_element_type=jnp.float32) + b_ref[...]
    out_ref[...] = jnp.where(y >= 0, y, 0.2 * y)


def _dense_update(msg, e, w, b):
    n = msg.shape[0]
    return pl.pallas_call(
        _dense_update_body,
        grid=(n // _ROW_BLK,),
        in_specs=[
            pl.BlockSpec((_ROW_BLK, D), lambda i: (i, 0)),
            pl.BlockSpec((_ROW_BLK, D), lambda i: (i, 0)),
            pl.BlockSpec((D, D), lambda i: (0, 0)),
            pl.BlockSpec((D,), lambda i: (0,)),
        ],
        out_specs=pl.BlockSpec((_ROW_BLK, D), lambda i: (i, 0)),
        out_shape=jax.ShapeDtypeStruct((n, D), jnp.float32),
    )(msg, e, w, b)


def _loss_body(u_ref, i_ref, lab_ref, logits_ref, loss_ref):
    step = pl.program_id(0)
    u = u_ref[...]
    nrm = jnp.sqrt(jnp.sum(u * u, axis=-1, keepdims=True))
    u = u / jnp.maximum(nrm, 1e-12)
    v = i_ref[...]
    logits = jnp.sum(u * v, axis=-1)
    logits_ref[...] = logits[None, :]
    lab = lab_ref[...][0]
    bce = jnp.maximum(logits, 0.0) - logits * lab + jnp.log1p(jnp.exp(-jnp.abs(logits)))
    reg = jnp.sum(u * u) + jnp.sum(v * v)
    part = jnp.sum(bce) / B + LAMBDA_REG * reg

    @pl.when(step == 0)
    def _init():
        loss_ref[0, 0] = 0.0

    loss_ref[0, 0] += part


def _loss(u_emb, i_emb, labels):
    blk = 2048
    logits, loss = pl.pallas_call(
        _loss_body,
        grid=(B // blk,),
        in_specs=[
            pl.BlockSpec((blk, D), lambda i: (i, 0)),
            pl.BlockSpec((blk, D), lambda i: (i, 0)),
            pl.BlockSpec((1, blk), lambda i: (0, i)),
        ],
        out_specs=[
            pl.BlockSpec((1, blk), lambda i: (0, i)),
            pl.BlockSpec(memory_space=pltpu.SMEM),
        ],
        out_shape=[
            jax.ShapeDtypeStruct((1, B), jnp.float32),
            jax.ShapeDtypeStruct((1, 1), jnp.float32),
        ],
    )(u_emb, i_emb, labels.reshape(1, B))
    return logits.reshape(B), loss[0, 0]


def kernel(uids, iids, labels, E_u_0, E_i_0, W0, b0, W1, b1, W2, b2,
           pos_idx, pos_val, neg_idx, neg_val, ii_idx, ii_val):
    keys, vals, counts = _bucket_kernel(
        pos_idx[0], pos_idx[1], pos_val,
        neg_idx[0], neg_idx[1], neg_val,
        ii_idx[0], ii_idx[1], ii_val)
    E_u = jnp.pad(E_u_0, ((0, NPAD - N_U), (0, 0)))
    E_i = jnp.pad(E_i_0, ((0, NPAD - N_I), (0, 0)))
    for W, b in ((W0, b0), (W1, b1), (W2, b2)):
        msg_u, msg_i = _spmm_kernel(keys, vals, counts, E_u, E_i)
        E_u = _dense_update(msg_u, E_u, W, b)
        E_i = _dense_update(msg_i, E_i, W, b)
    u_emb, i_emb = _gather_kernel(E_u, E_i, uids, iids)
    logits, loss = _loss(u_emb, i_emb, labels)
    return (loss, logits)
